# Initial kernel scaffold; baseline (speedup 1.0000x reference)
import jax
import jax.numpy as jnp
from jax.experimental import pallas as pl


def kernel(x, edge_index, edge_attr, W_e, b_e, W1, b1, W2, b2, ln_w, ln_b):
    raise NotImplementedError("write your pallas kernel here")


# speedup vs baseline: 1.0053x; 1.0053x over previous
import jax
import jax.numpy as jnp
from jax import lax
from jax.experimental import pallas as pl
from jax.experimental.pallas import tpu as pltpu

N, E, D = 10000, 320000, 128


def _final_body(x_ref, aggr_ref, W1_ref, b1_ref, W2_ref, b2_ref,
                lnw_ref, lnb_ref, out_ref):
    x = x_ref[...]
    h0 = x + aggr_ref[...]
    dn = (((1,), (1,)), ((), ()))
    h = lax.dot_general(h0, W1_ref[...], dn,
                        preferred_element_type=jnp.float32) + b1_ref[...]
    h = jnp.maximum(h, 0.0)
    h = lax.dot_general(h, W2_ref[...], dn,
                        preferred_element_type=jnp.float32) + b2_ref[...]
    h = h + x
    mean = jnp.mean(h)
    var = jnp.mean((h - mean) ** 2)
    h = (h - mean) / (jnp.sqrt(var) + 1e-5)
    h = h * lnw_ref[...] + lnb_ref[...]
    h = h * jax.nn.sigmoid(h)
    out_ref[...] = jnp.nan_to_num(h)


def _final_stage(x, aggr, W1, b1, W2, b2, ln_w, ln_b):
    return pl.pallas_call(
        _final_body,
        out_shape=jax.ShapeDtypeStruct((N, D), jnp.float32),
    )(x, aggr, W1, b1.reshape(1, D), W2, b2.reshape(1, D),
      ln_w.reshape(1, D), ln_b.reshape(1, D))


def kernel(x, edge_index, edge_attr, W_e, b_e, W1, b1, W2, b2, ln_w, ln_b):
    src = edge_index[0]
    dst = edge_index[1]
    e = edge_attr @ W_e.T + b_e
    msg = jax.nn.relu(jnp.take(x, src, axis=0) + e)
    aggr = jax.ops.segment_sum(msg, dst, num_segments=N)
    return _final_stage(x, aggr, W1, b1, W2, b2, ln_w, ln_b)


# speedup vs baseline: 3.6882x; 3.6688x over previous
import functools

import jax
import jax.numpy as jnp
from jax import lax
from jax.experimental import pallas as pl
from jax.experimental.pallas import tpu as pltpu
from jax.experimental.pallas import tpu_sc as plsc

N, E, D = 10000, 320000, 128

_NC, _NS, _L = 2, 16, 16
_NW = _NC * _NS
_CH = 128
_NCHUNKS = E // _CH
_CPW = -(-_NCHUNKS // _NW)
_NPAD = 10240
_RPT = _NPAD // _NS
_RCP = 128
_EBLK = 6400


def _edge_mlp_body(ea_ref, We_ref, be_ref, out_ref):
    dn = (((1,), (1,)), ((), ()))
    out_ref[...] = lax.dot_general(
        ea_ref[...], We_ref[...], dn,
        preferred_element_type=jnp.float32) + be_ref[...]


def _edge_mlp(edge_attr, W_e, b_e):
    return pl.pallas_call(
        _edge_mlp_body,
        grid=(E // _EBLK,),
        in_specs=[
            pl.BlockSpec((_EBLK, D), lambda i: (i, 0)),
            pl.BlockSpec((D, D), lambda i: (0, 0)),
            pl.BlockSpec((1, D), lambda i: (0, 0)),
        ],
        out_specs=pl.BlockSpec((_EBLK, D), lambda i: (i, 0)),
        out_shape=jax.ShapeDtypeStruct((E, D), jnp.float32),
    )(edge_attr, W_e, b_e.reshape(1, D))


def _mp_body(x_hbm, src_hbm, dst_hbm, e_hbm, out_hbm,
             idx_s, idx_d, msg, stage, aggr_sh, sem):
    cid = lax.axis_index("c")
    sid = lax.axis_index("s")
    wid = cid * _NS + sid

    def zrow(r, carry):
        for j in range(D // _L):
            stage[r, pl.ds(j * _L, _L)] = jnp.zeros((_L,), jnp.float32)
        return carry
    lax.fori_loop(0, _RCP, zrow, 0)
    for k in range(_RPT // _RCP):
        r0 = pl.multiple_of(sid * _RPT + k * _RCP, _RCP)
        pltpu.sync_copy(stage.at[pl.ds(0, _RCP)], aggr_sh.at[pl.ds(r0, _RCP)])
    plsc.subcore_barrier()

    def chunk(i, carry):
        g = i * _NW + wid

        @pl.when(g < _NCHUNKS)
        def _():
            base = pl.multiple_of(g * _CH, _CH)
            pltpu.sync_copy(src_hbm.at[pl.ds(base, _CH)], idx_s)
            pltpu.sync_copy(dst_hbm.at[pl.ds(base, _CH)], idx_d)
            pltpu.sync_copy(e_hbm.at[pl.ds(base, _CH)], msg)
            pltpu.async_copy(x_hbm.at[idx_s], msg, sem, add=True).wait()

            def rrow(r, c2):
                for j in range(D // _L):
                    v = msg[r, pl.ds(j * _L, _L)]
                    msg[r, pl.ds(j * _L, _L)] = jnp.maximum(v, 0.0)
                return c2
            lax.fori_loop(0, _CH, rrow, 0)
            pltpu.sync_copy(msg, aggr_sh.at[idx_d], add=True)
        return carry
    lax.fori_loop(0, _CPW, chunk, 0)

    plsc.subcore_barrier()
    for k in range(_RPT // _RCP):
        r0 = pl.multiple_of(sid * _RPT + k * _RCP, _RCP)
        pltpu.sync_copy(aggr_sh.at[pl.ds(r0, _RCP)], stage.at[pl.ds(0, _RCP)])
        pltpu.sync_copy(stage.at[pl.ds(0, _RCP)], out_hbm.at[cid, pl.ds(r0, _RCP)])


def _message_passing(x, src, dst, e):
    mesh = plsc.VectorSubcoreMesh(core_axis_name="c", subcore_axis_name="s")
    f = functools.partial(
        pl.kernel,
        out_type=jax.ShapeDtypeStruct((_NC, _NPAD, D), jnp.float32),
        mesh=mesh,
        scratch_types=[
            pltpu.VMEM((_CH,), jnp.int32),
            pltpu.VMEM((_CH,), jnp.int32),
            pltpu.VMEM((_CH, D), jnp.float32),
            pltpu.VMEM((_RCP, D), jnp.float32),
            pltpu.VMEM_SHARED((_NPAD, D), jnp.float32),
            pltpu.SemaphoreType.DMA,
        ],
    )(_mp_body)
    return f(x, src, dst, e)


def _final_body(x_ref, p_ref, W1_ref, b1_ref, W2_ref, b2_ref,
                lnw_ref, lnb_ref, out_ref):
    x = x_ref[...]
    p = p_ref[...]
    h0 = x + p[0, :N] + p[1, :N]
    dn = (((1,), (1,)), ((), ()))
    h = lax.dot_general(h0, W1_ref[...], dn,
                        preferred_element_type=jnp.float32) + b1_ref[...]
    h = jnp.maximum(h, 0.0)
    h = lax.dot_general(h, W2_ref[...], dn,
                        preferred_element_type=jnp.float32) + b2_ref[...]
    h = h + x
    mean = jnp.mean(h)
    var = jnp.mean((h - mean) ** 2)
    h = (h - mean) / (jnp.sqrt(var) + 1e-5)
    h = h * lnw_ref[...] + lnb_ref[...]
    h = h * jax.nn.sigmoid(h)
    out_ref[...] = jnp.nan_to_num(h)


def _final_stage(x, partials, W1, b1, W2, b2, ln_w, ln_b):
    return pl.pallas_call(
        _final_body,
        out_shape=jax.ShapeDtypeStruct((N, D), jnp.float32),
    )(x, partials, W1, b1.reshape(1, D), W2, b2.reshape(1, D),
      ln_w.reshape(1, D), ln_b.reshape(1, D))


def kernel(x, edge_index, edge_attr, W_e, b_e, W1, b1, W2, b2, ln_w, ln_b):
    src = edge_index[0]
    dst = edge_index[1]
    e = _edge_mlp(edge_attr, W_e, b_e)
    partials = _message_passing(x, src, dst, e)
    return _final_stage(x, partials, W1, b1, W2, b2, ln_w, ln_b)


# speedup vs baseline: 4.8433x; 1.3132x over previous
import functools

import jax
import jax.numpy as jnp
from jax import lax
from jax.experimental import pallas as pl
from jax.experimental.pallas import tpu as pltpu
from jax.experimental.pallas import tpu_sc as plsc

N, E, D = 10000, 320000, 128

_NC, _NS, _L = 2, 16, 16
_NW = _NC * _NS
_CH = 64
_NCHUNKS = E // _CH
_CPW = 160
_CPAD = _NW * _CPW
_EPAD = _CPAD * _CH
_NBUF = 4
_NPAD = 10112
_RPT = _NPAD // _NS
_EBLK = 6400


def _edge_mlp_body(ea_ref, We_ref, be_ref, out_ref):
    dn = (((1,), (1,)), ((), ()))
    out_ref[...] = lax.dot_general(
        ea_ref[...], We_ref[...], dn,
        preferred_element_type=jnp.float32) + be_ref[...]


def _edge_mlp(edge_attr, W_e, b_e):
    return pl.pallas_call(
        _edge_mlp_body,
        grid=(E // _EBLK,),
        in_specs=[
            pl.BlockSpec((_EBLK, D), lambda i: (i, 0)),
            pl.BlockSpec((D, D), lambda i: (0, 0)),
            pl.BlockSpec((1, D), lambda i: (0, 0)),
        ],
        out_specs=pl.BlockSpec((_EBLK, D), lambda i: (i, 0)),
        out_shape=jax.ShapeDtypeStruct((E, D), jnp.float32),
    )(edge_attr, W_e, b_e.reshape(1, D))


_STRIPE = [(k * _CH, _CH) for k in range(_RPT // _CH)]
if _RPT % _CH:
    _STRIPE.append(((_RPT // _CH) * _CH, _RPT % _CH))


def _mp_body(x_hbm, src_hbm, dst_hbm, e_hbm, out_hbm,
             idx_s, idx_d, msg, aggr_sh, sem_e, sem_g, sem_i, sem_sc):
    cid = lax.axis_index("c")
    sid = lax.axis_index("s")
    wid = cid * _NS + sid
    c0 = wid * _CPW

    def zrow(r, carry):
        for j in range(D // _L):
            msg[0][r, pl.ds(j * _L, _L)] = jnp.zeros((_L,), jnp.float32)
        return carry
    lax.fori_loop(0, _CH, zrow, 0)
    for off, nr in _STRIPE:
        r0 = pl.multiple_of(sid * _RPT + off, 8)
        pltpu.sync_copy(msg[0].at[pl.ds(0, nr)], aggr_sh.at[pl.ds(r0, nr)])
    plsc.subcore_barrier()

    def _ebase(c):
        g = c0 + c
        g = jnp.where(g < _NCHUNKS, g, 0)
        return pl.multiple_of(g * _CH, _CH)

    def _ibase(c):
        return pl.multiple_of((c0 + c) * _CH, _CH)

    def _i_issue(b, c):
        base = _ibase(c)
        pltpu.async_copy(src_hbm.at[pl.ds(base, _CH)], idx_s[b], sem_i[b])
        pltpu.async_copy(dst_hbm.at[pl.ds(base, _CH)], idx_d[b], sem_i[b])

    def _i_wait(b, c):
        base = _ibase(c)
        pltpu.make_async_copy(src_hbm.at[pl.ds(base, _CH)], idx_s[b],
                              sem_i[b]).wait()
        pltpu.make_async_copy(dst_hbm.at[pl.ds(base, _CH)], idx_d[b],
                              sem_i[b]).wait()

    def _e_issue(b, c):
        pltpu.async_copy(e_hbm.at[pl.ds(_ebase(c), _CH)], msg[b], sem_e[b])

    def _e_wait(b, c):
        pltpu.make_async_copy(e_hbm.at[pl.ds(_ebase(c), _CH)], msg[b],
                              sem_e[b]).wait()

    def _g_issue(b):
        pltpu.async_copy(x_hbm.at[idx_s[b]], msg[b], sem_g[b], add=True)

    def _g_wait(b):
        pltpu.make_async_copy(x_hbm.at[idx_s[b]], msg[b], sem_g[b]).wait()

    def _sc_issue(b):
        pltpu.async_copy(msg[b], aggr_sh.at[idx_d[b]], sem_sc[b], add=True)

    def _sc_wait(b):
        pltpu.make_async_copy(msg[b], aggr_sh.at[idx_d[b]], sem_sc[b]).wait()

    _i_issue(0, 0)
    _e_issue(0, 0)
    _i_issue(1, 1)
    _e_issue(1, 1)
    _i_wait(0, 0)
    _e_wait(0, 0)
    _g_issue(0)

    def body(t, carry):
        for b in range(_NBUF):
            c = t * _NBUF + b
            _g_wait(b)

            def rrow(r, c2):
                for j in range(D // _L):
                    v = msg[b][r, pl.ds(j * _L, _L)]
                    msg[b][r, pl.ds(j * _L, _L)] = jnp.maximum(v, 0.0)
                return c2
            lax.fori_loop(0, _CH, rrow, 0)
            _sc_issue(b)

            @pl.when(c >= 2)
            def _():
                _sc_wait((b + 2) % _NBUF)

            @pl.when(c + 2 < _CPW)
            def _():
                _i_issue((b + 2) % _NBUF, c + 2)
                _e_issue((b + 2) % _NBUF, c + 2)

            @pl.when(c + 1 < _CPW)
            def _():
                _i_wait((b + 1) % _NBUF, c + 1)
                _e_wait((b + 1) % _NBUF, c + 1)
                _g_issue((b + 1) % _NBUF)
        return carry
    lax.fori_loop(0, _CPW // _NBUF, body, 0)
    _sc_wait((_CPW - 2) % _NBUF)
    _sc_wait((_CPW - 1) % _NBUF)

    plsc.subcore_barrier()
    for off, nr in _STRIPE:
        r0 = pl.multiple_of(sid * _RPT + off, 8)
        pltpu.sync_copy(aggr_sh.at[pl.ds(r0, nr)], msg[0].at[pl.ds(0, nr)])
        pltpu.sync_copy(msg[0].at[pl.ds(0, nr)], out_hbm.at[cid, pl.ds(r0, nr)])


def _message_passing(x, src, dst, e):
    mesh = plsc.VectorSubcoreMesh(core_axis_name="c", subcore_axis_name="s")
    f = functools.partial(
        pl.kernel,
        out_type=jax.ShapeDtypeStruct((_NC, _NPAD, D), jnp.float32),
        mesh=mesh,
        scratch_types=[
            [pltpu.VMEM((_CH,), jnp.int32)] * _NBUF,
            [pltpu.VMEM((_CH,), jnp.int32)] * _NBUF,
            [pltpu.VMEM((_CH, D), jnp.float32)] * _NBUF,
            pltpu.VMEM_SHARED((_NPAD, D), jnp.float32),
            [pltpu.SemaphoreType.DMA] * _NBUF,
            [pltpu.SemaphoreType.DMA] * _NBUF,
            [pltpu.SemaphoreType.DMA] * _NBUF,
            [pltpu.SemaphoreType.DMA] * _NBUF,
        ],
    )(_mp_body)
    return f(x, src, dst, e)


def _final_body(x_ref, p_ref, W1_ref, b1_ref, W2_ref, b2_ref,
                lnw_ref, lnb_ref, out_ref):
    x = x_ref[...]
    p = p_ref[...]
    h0 = x + p[0, :N] + p[1, :N]
    dn = (((1,), (1,)), ((), ()))
    h = lax.dot_general(h0, W1_ref[...], dn,
                        preferred_element_type=jnp.float32) + b1_ref[...]
    h = jnp.maximum(h, 0.0)
    h = lax.dot_general(h, W2_ref[...], dn,
                        preferred_element_type=jnp.float32) + b2_ref[...]
    h = h + x
    mean = jnp.mean(h)
    var = jnp.mean((h - mean) ** 2)
    h = (h - mean) / (jnp.sqrt(var) + 1e-5)
    h = h * lnw_ref[...] + lnb_ref[...]
    h = h * jax.nn.sigmoid(h)
    out_ref[...] = jnp.nan_to_num(h)


def _final_stage(x, partials, W1, b1, W2, b2, ln_w, ln_b):
    return pl.pallas_call(
        _final_body,
        out_shape=jax.ShapeDtypeStruct((N, D), jnp.float32),
    )(x, partials, W1, b1.reshape(1, D), W2, b2.reshape(1, D),
      ln_w.reshape(1, D), ln_b.reshape(1, D))


def kernel(x, edge_index, edge_attr, W_e, b_e, W1, b1, W2, b2, ln_w, ln_b):
    src = edge_index[0]
    dst = edge_index[1]
    npad = _EPAD - E
    pad_iota = jnp.arange(npad, dtype=jnp.int32)
    srcp = jnp.concatenate([src, pad_iota % N])
    dstp = jnp.concatenate([dst, N + pad_iota % (_NPAD - N)])
    e = _edge_mlp(edge_attr, W_e, b_e)
    partials = _message_passing(x, srcp, dstp, e)
    return _final_stage(x, partials, W1, b1, W2, b2, ln_w, ln_b)


# speedup vs baseline: 5.9454x; 1.2276x over previous
import functools

import jax
import jax.numpy as jnp
from jax import lax
from jax.experimental import pallas as pl
from jax.experimental.pallas import tpu as pltpu
from jax.experimental.pallas import tpu_sc as plsc

N, E, D = 10000, 320000, 128

_NC, _NS, _L = 2, 16, 16
_NW = _NC * _NS
_CH = 64
_NCHUNKS = E // _CH
_CPW = 160
_CPAD = _NW * _CPW
_EPAD = _CPAD * _CH
_NBUF = 5
_NPAD = 10112
_RPT = _NPAD // _NS
_EBLK = 6400


def _edge_mlp_body(ea_ref, We_ref, be_ref, out_ref):
    dn = (((1,), (1,)), ((), ()))
    out_ref[...] = lax.dot_general(
        ea_ref[...], We_ref[...], dn,
        preferred_element_type=jnp.float32) + be_ref[...]


def _edge_mlp(edge_attr, W_e, b_e):
    return pl.pallas_call(
        _edge_mlp_body,
        grid=(E // _EBLK,),
        in_specs=[
            pl.BlockSpec((_EBLK, D), lambda i: (i, 0)),
            pl.BlockSpec((D, D), lambda i: (0, 0)),
            pl.BlockSpec((1, D), lambda i: (0, 0)),
        ],
        out_specs=pl.BlockSpec((_EBLK, D), lambda i: (i, 0)),
        out_shape=jax.ShapeDtypeStruct((E, D), jnp.float32),
    )(edge_attr, W_e, b_e.reshape(1, D))


_STRIPE = [(k * _CH, _CH) for k in range(_RPT // _CH)]
if _RPT % _CH:
    _STRIPE.append(((_RPT // _CH) * _CH, _RPT % _CH))


def _mp_body(x_hbm, src_hbm, dst_hbm, e_hbm, out_hbm,
             idx_s, idx_d, msg, aggr_sh, sem_e, sem_g, sem_i, sem_sc):
    cid = lax.axis_index("c")
    sid = lax.axis_index("s")
    wid = cid * _NS + sid
    c0 = wid * _CPW

    def zrow(r, carry):
        for j in range(D // _L):
            msg[0][r, pl.ds(j * _L, _L)] = jnp.zeros((_L,), jnp.float32)
        return carry
    lax.fori_loop(0, _CH, zrow, 0)
    for off, nr in _STRIPE:
        r0 = pl.multiple_of(sid * _RPT + off, 8)
        pltpu.sync_copy(msg[0].at[pl.ds(0, nr)], aggr_sh.at[pl.ds(r0, nr)])
    plsc.subcore_barrier()

    def _ebase(c):
        g = c0 + c
        g = jnp.where(g < _NCHUNKS, g, 0)
        return pl.multiple_of(g * _CH, _CH)

    def _ibase(c):
        return pl.multiple_of((c0 + c) * _CH, _CH)

    def _i_issue(b, c):
        base = _ibase(c)
        pltpu.async_copy(src_hbm.at[pl.ds(base, _CH)], idx_s[b], sem_i[b])
        pltpu.async_copy(dst_hbm.at[pl.ds(base, _CH)], idx_d[b], sem_i[b])

    def _i_wait(b, c):
        base = _ibase(c)
        pltpu.make_async_copy(src_hbm.at[pl.ds(base, _CH)], idx_s[b],
                              sem_i[b]).wait()
        pltpu.make_async_copy(dst_hbm.at[pl.ds(base, _CH)], idx_d[b],
                              sem_i[b]).wait()

    def _e_issue(b, c):
        pltpu.async_copy(e_hbm.at[pl.ds(_ebase(c), _CH)], msg[b], sem_e[b])

    def _e_wait(b, c):
        pltpu.make_async_copy(e_hbm.at[pl.ds(_ebase(c), _CH)], msg[b],
                              sem_e[b]).wait()

    def _g_issue(b):
        pltpu.async_copy(x_hbm.at[idx_s[b]], msg[b], sem_g[b], add=True)

    def _g_wait(b):
        pltpu.make_async_copy(x_hbm.at[idx_s[b]], msg[b], sem_g[b]).wait()

    def _sc_issue(b):
        pltpu.async_copy(msg[b], aggr_sh.at[idx_d[b]], sem_sc[b], add=True)

    def _sc_wait(b):
        pltpu.make_async_copy(msg[b], aggr_sh.at[idx_d[b]], sem_sc[b]).wait()

    for k in range(3):
        _i_issue(k, k)
        _e_issue(k, k)
    for k in range(2):
        _i_wait(k, k)
        _e_wait(k, k)
        _g_issue(k)

    def body(t, carry):
        for b in range(_NBUF):
            c = t * _NBUF + b
            _g_wait(b)

            def rrow(r, c2):
                for j in range(D // _L):
                    v = msg[b][r, pl.ds(j * _L, _L)]
                    msg[b][r, pl.ds(j * _L, _L)] = jnp.maximum(v, 0.0)
                return c2
            lax.fori_loop(0, _CH, rrow, 0)
            _sc_issue(b)

            @pl.when(c >= 2)
            def _():
                _sc_wait((b + 3) % _NBUF)

            @pl.when(c + 3 < _CPW)
            def _():
                _i_issue((b + 3) % _NBUF, c + 3)
                _e_issue((b + 3) % _NBUF, c + 3)

            @pl.when(c + 2 < _CPW)
            def _():
                _i_wait((b + 2) % _NBUF, c + 2)
                _e_wait((b + 2) % _NBUF, c + 2)
                _g_issue((b + 2) % _NBUF)
        return carry
    lax.fori_loop(0, _CPW // _NBUF, body, 0)
    _sc_wait((_CPW - 2) % _NBUF)
    _sc_wait((_CPW - 1) % _NBUF)

    plsc.subcore_barrier()
    for off, nr in _STRIPE:
        r0 = pl.multiple_of(sid * _RPT + off, 8)
        pltpu.sync_copy(aggr_sh.at[pl.ds(r0, nr)], msg[0].at[pl.ds(0, nr)])
        pltpu.sync_copy(msg[0].at[pl.ds(0, nr)], out_hbm.at[cid, pl.ds(r0, nr)])


def _message_passing(x, src, dst, e):
    mesh = plsc.VectorSubcoreMesh(core_axis_name="c", subcore_axis_name="s")
    f = functools.partial(
        pl.kernel,
        out_type=jax.ShapeDtypeStruct((_NC, _NPAD, D), jnp.float32),
        mesh=mesh,
        scratch_types=[
            [pltpu.VMEM((_CH,), jnp.int32)] * _NBUF,
            [pltpu.VMEM((_CH,), jnp.int32)] * _NBUF,
            [pltpu.VMEM((_CH, D), jnp.float32)] * _NBUF,
            pltpu.VMEM_SHARED((_NPAD, D), jnp.float32),
            [pltpu.SemaphoreType.DMA] * _NBUF,
            [pltpu.SemaphoreType.DMA] * _NBUF,
            [pltpu.SemaphoreType.DMA] * _NBUF,
            [pltpu.SemaphoreType.DMA] * _NBUF,
        ],
    )(_mp_body)
    return f(x, src, dst, e)


def _final_body(x_ref, p_ref, W1_ref, b1_ref, W2_ref, b2_ref,
                lnw_ref, lnb_ref, out_ref):
    x = x_ref[...]
    p = p_ref[...]
    h0 = x + p[0, :N] + p[1, :N]
    dn = (((1,), (1,)), ((), ()))
    h = lax.dot_general(h0, W1_ref[...], dn,
                        preferred_element_type=jnp.float32) + b1_ref[...]
    h = jnp.maximum(h, 0.0)
    h = lax.dot_general(h, W2_ref[...], dn,
                        preferred_element_type=jnp.float32) + b2_ref[...]
    h = h + x
    mean = jnp.mean(h)
    var = jnp.mean((h - mean) ** 2)
    h = (h - mean) / (jnp.sqrt(var) + 1e-5)
    h = h * lnw_ref[...] + lnb_ref[...]
    h = h * jax.nn.sigmoid(h)
    out_ref[...] = jnp.nan_to_num(h)


def _final_stage(x, partials, W1, b1, W2, b2, ln_w, ln_b):
    return pl.pallas_call(
        _final_body,
        out_shape=jax.ShapeDtypeStruct((N, D), jnp.float32),
    )(x, partials, W1, b1.reshape(1, D), W2, b2.reshape(1, D),
      ln_w.reshape(1, D), ln_b.reshape(1, D))


def kernel(x, edge_index, edge_attr, W_e, b_e, W1, b1, W2, b2, ln_w, ln_b):
    src = edge_index[0]
    dst = edge_index[1]
    npad = _EPAD - E
    pad_iota = jnp.arange(npad, dtype=jnp.int32)
    srcp = jnp.concatenate([src, pad_iota % N])
    dstp = jnp.concatenate([dst, N + pad_iota % (_NPAD - N)])
    e = _edge_mlp(edge_attr, W_e, b_e)
    partials = _message_passing(x, srcp, dstp, e)
    return _final_stage(x, partials, W1, b1, W2, b2, ln_w, ln_b)
